# Initial kernel scaffold; baseline (speedup 1.0000x reference)
import jax
import jax.numpy as jnp
from jax.experimental import pallas as pl


def kernel(x, edge_index, W1, b1, W2, b2, W3, b3):
    raise NotImplementedError("write your pallas kernel here")


# speedup vs baseline: 16.5087x; 16.5087x over previous
import functools

import jax
import jax.numpy as jnp
from jax import lax
from jax.experimental import pallas as pl
from jax.experimental.pallas import tpu as pltpu
from jax.experimental.pallas import tpu_sc as plsc

NC = 2
NS = 16
NW = NC * NS
K_EDGE = 128


def _mesh():
  return plsc.VectorSubcoreMesh(
      core_axis_name="c", subcore_axis_name="s", num_cores=NC,
      num_subcores=NS)


def _deg_body(npt, n_pad, dst_i, ones, zeros16, out, dst_v, ones_v, acc):
  cid = lax.axis_index("c")
  sid = lax.axis_index("s")
  wid = sid * NC + cid
  rpt = n_pad // NS
  sl = pl.ds(sid * rpt, rpt)
  pltpu.sync_copy(dst_i.at[wid], dst_v)
  pltpu.sync_copy(ones, ones_v)
  pltpu.sync_copy(zeros16.at[sl], acc.at[sl])
  plsc.subcore_barrier()

  @pl.loop(0, npt)
  def _(j):
    pltpu.sync_copy(ones_v, acc.at[dst_v.at[j]], add=True)

  plsc.subcore_barrier()
  pltpu.sync_copy(acc.at[sl], out.at[cid].at[sl])


def _make_deg(npt, n_pad):
  return pl.kernel(
      functools.partial(_deg_body, npt, n_pad),
      out_type=jax.ShapeDtypeStruct((NC, n_pad, 16), jnp.float32),
      mesh=_mesh(),
      compiler_params=pltpu.CompilerParams(use_tc_tiling_on_sc=False),
      scratch_types=[
          pltpu.VMEM((npt, K_EDGE), jnp.int32),
          pltpu.VMEM((K_EDGE, 16), jnp.float32),
          pltpu.VMEM_SHARED((n_pad, 16), jnp.float32),
      ],
  )


def _agg_body(npt, n_pad, d, table, src_i, dst_i, zeros, out,
              src_v, dst_v, rows, sem, acc):
  cid = lax.axis_index("c")
  sid = lax.axis_index("s")
  wid = sid * NC + cid
  rpt = n_pad // NS
  sl = pl.ds(sid * rpt, rpt)
  pltpu.sync_copy(src_i.at[wid], src_v)
  pltpu.sync_copy(dst_i.at[wid], dst_v)
  pltpu.sync_copy(zeros.at[sl], acc.at[sl])
  plsc.subcore_barrier()

  @pl.loop(0, npt)
  def _(j):
    pltpu.async_copy(table.at[src_v.at[j]], rows, sem).wait()
    pltpu.sync_copy(rows, acc.at[dst_v.at[j]], add=True)

  plsc.subcore_barrier()
  pltpu.sync_copy(acc.at[sl], out.at[cid].at[sl])


def _make_agg(npt, n_pad, d):
  return pl.kernel(
      functools.partial(_agg_body, npt, n_pad, d),
      out_type=jax.ShapeDtypeStruct((NC, n_pad, d), jnp.float32),
      mesh=_mesh(),
      compiler_params=pltpu.CompilerParams(use_tc_tiling_on_sc=False),
      scratch_types=[
          pltpu.VMEM((npt, K_EDGE), jnp.int32),
          pltpu.VMEM((npt, K_EDGE), jnp.int32),
          pltpu.VMEM((K_EDGE, d), jnp.float32),
          pltpu.SemaphoreType.DMA,
          pltpu.VMEM_SHARED((n_pad, d), jnp.float32),
      ],
  )


def _s_block(degp, n, r0):
  dsum = degp[0, :, 0:1] + degp[1, :, 0:1]
  s = jnp.where(dsum > 0, lax.rsqrt(jnp.maximum(dsum, 1e-12)), 0.0)
  rows = r0 + lax.broadcasted_iota(jnp.int32, s.shape, 0)
  return jnp.where(rows < n, s, 0.0)


def _lin_first_body(n, r, x_ref, w_ref, degp_ref, o_ref):
  i = pl.program_id(0)
  s = _s_block(degp_ref[...], n, i * r)
  o_ref[...] = s * jnp.dot(x_ref[...], w_ref[...],
                           preferred_element_type=jnp.float32)


def _lin_mid_body(n, r, p_ref, b_ref, w_ref, degp_ref, o_ref):
  i = pl.program_id(0)
  s = _s_block(degp_ref[...], n, i * r)
  z = s * (p_ref[0] + p_ref[1]) + b_ref[...]
  a = jnp.maximum(z, 0.0)
  o_ref[...] = s * jnp.dot(a, w_ref[...], preferred_element_type=jnp.float32)


def _final_body(n, r, nvalid, p_ref, b_ref, degp_ref, o_ref):
  i = pl.program_id(0)
  s = _s_block(degp_ref[...], n, i * r)
  z = s * (p_ref[0] + p_ref[1]) + b_ref[...]
  col = lax.broadcasted_iota(jnp.int32, z.shape, 1)
  valid = col < nvalid
  zm = jnp.where(valid, z, -jnp.inf)
  m = jnp.max(zm, axis=1, keepdims=True)
  e = jnp.where(valid, jnp.exp(zm - m), 0.0)
  lse = jnp.log(jnp.sum(e, axis=1, keepdims=True))
  o_ref[...] = z - m - lse


_R = 512


def _tc_first(n, n_pad, din, dout):
  grid = n_pad // _R
  return pl.pallas_call(
      functools.partial(_lin_first_body, n, _R),
      grid=(grid,),
      in_specs=[
          pl.BlockSpec((_R, din), lambda i: (i, 0)),
          pl.BlockSpec((din, dout), lambda i: (0, 0)),
          pl.BlockSpec((NC, _R, 16), lambda i: (0, i, 0)),
      ],
      out_specs=pl.BlockSpec((_R, dout), lambda i: (i, 0)),
      out_shape=jax.ShapeDtypeStruct((n_pad, dout), jnp.float32),
  )


def _tc_mid(n, n_pad, din, dout):
  grid = n_pad // _R
  return pl.pallas_call(
      functools.partial(_lin_mid_body, n, _R),
      grid=(grid,),
      in_specs=[
          pl.BlockSpec((NC, _R, din), lambda i: (0, i, 0)),
          pl.BlockSpec((1, din), lambda i: (0, 0)),
          pl.BlockSpec((din, dout), lambda i: (0, 0)),
          pl.BlockSpec((NC, _R, 16), lambda i: (0, i, 0)),
      ],
      out_specs=pl.BlockSpec((_R, dout), lambda i: (i, 0)),
      out_shape=jax.ShapeDtypeStruct((n_pad, dout), jnp.float32),
  )


def _tc_final(n, n_pad, d, nvalid):
  grid = n_pad // _R
  return pl.pallas_call(
      functools.partial(_final_body, n, _R, nvalid),
      grid=(grid,),
      in_specs=[
          pl.BlockSpec((NC, _R, d), lambda i: (0, i, 0)),
          pl.BlockSpec((1, d), lambda i: (0, 0)),
          pl.BlockSpec((NC, _R, 16), lambda i: (0, i, 0)),
      ],
      out_specs=pl.BlockSpec((_R, d), lambda i: (i, 0)),
      out_shape=jax.ShapeDtypeStruct((n_pad, d), jnp.float32),
  )


def kernel(x, edge_index, W1, b1, W2, b2, W3, b3):
  n, in_dim = x.shape
  e = edge_index.shape[1]
  h1 = W1.shape[1]
  h2 = W2.shape[1]
  dout = W3.shape[1]
  dout_p = ((dout + 15) // 16) * 16

  n_pad = ((n + _R - 1) // _R) * _R
  e_tot = e + n
  npt = (e_tot + NW * K_EDGE - 1) // (NW * K_EDGE)
  e_pad = NW * npt * K_EDGE

  loop = jnp.arange(n, dtype=jnp.int32)
  src = jnp.concatenate([edge_index[0].astype(jnp.int32), loop])
  dst = jnp.concatenate([edge_index[1].astype(jnp.int32), loop])
  pad = jnp.full((e_pad - e_tot,), n, dtype=jnp.int32)
  src_i = jnp.concatenate([src, pad]).reshape(NW, npt, K_EDGE)
  dst_i = jnp.concatenate([dst, pad]).reshape(NW, npt, K_EDGE)

  x_pad = jnp.pad(x, ((0, n_pad - n), (0, 0)))
  w3p = jnp.pad(W3, ((0, 0), (0, dout_p - dout)))
  b1r = b1.reshape(1, h1)
  b2r = b2.reshape(1, h2)
  b3r = jnp.pad(b3, (0, dout_p - dout)).reshape(1, dout_p)

  ones16 = jnp.ones((K_EDGE, 16), jnp.float32)
  zeros16 = jnp.zeros((n_pad, 16), jnp.float32)

  degp = _make_deg(npt, n_pad)(dst_i, ones16, zeros16)

  t1 = _tc_first(n, n_pad, in_dim, h1)(x_pad, W1, degp)
  p1 = _make_agg(npt, n_pad, h1)(t1, src_i, dst_i,
                                 jnp.zeros((n_pad, h1), jnp.float32))
  t2 = _tc_mid(n, n_pad, h1, h2)(p1, b1r, W2, degp)
  p2 = _make_agg(npt, n_pad, h2)(t2, src_i, dst_i,
                                 jnp.zeros((n_pad, h2), jnp.float32))
  t3 = _tc_mid(n, n_pad, h2, dout_p)(p2, b2r, w3p, degp)
  p3 = _make_agg(npt, n_pad, dout_p)(t3, src_i, dst_i,
                                     jnp.zeros((n_pad, dout_p), jnp.float32))
  o = _tc_final(n, n_pad, dout_p, dout)(p3, b3r, degp)
  return o[:n, :dout]


# speedup vs baseline: 17.2615x; 1.0456x over previous
import functools

import jax
import jax.numpy as jnp
from jax import lax
from jax.experimental import pallas as pl
from jax.experimental.pallas import tpu as pltpu
from jax.experimental.pallas import tpu_sc as plsc

NC = 2
NS = 16
NW = NC * NS
K_EDGE = 96


def _mesh():
  return plsc.VectorSubcoreMesh(
      core_axis_name="c", subcore_axis_name="s", num_cores=NC,
      num_subcores=NS)


def _deg_body(npt, n_pad, dst_i, ones, zeros16, out, dst_v, ones_v, acc):
  cid = lax.axis_index("c")
  sid = lax.axis_index("s")
  wid = sid * NC + cid
  rpt = n_pad // NS
  sl = pl.ds(sid * rpt, rpt)
  pltpu.sync_copy(dst_i.at[wid], dst_v)
  pltpu.sync_copy(ones, ones_v)
  pltpu.sync_copy(zeros16.at[sl], acc.at[sl])
  plsc.subcore_barrier()

  @pl.loop(0, npt)
  def _(j):
    pltpu.sync_copy(ones_v, acc.at[dst_v.at[j]], add=True)

  plsc.subcore_barrier()
  pltpu.sync_copy(acc.at[sl], out.at[cid].at[sl])


def _make_deg(npt, n_pad):
  return pl.kernel(
      functools.partial(_deg_body, npt, n_pad),
      out_type=jax.ShapeDtypeStruct((NC, n_pad, 16), jnp.float32),
      mesh=_mesh(),
      compiler_params=pltpu.CompilerParams(use_tc_tiling_on_sc=False),
      scratch_types=[
          pltpu.VMEM((npt, K_EDGE), jnp.int32),
          pltpu.VMEM((K_EDGE, 16), jnp.float32),
          pltpu.VMEM_SHARED((n_pad, 16), jnp.float32),
      ],
  )


NBUF = 2


def _agg_body(npt, n_pad, d, table, src_i, dst_i, zeros, out,
              src_v, dst_v, rows0, rows1, sem0, sem1, acc):
  cid = lax.axis_index("c")
  sid = lax.axis_index("s")
  wid = sid * NC + cid
  rpt = n_pad // NS
  sl = pl.ds(sid * rpt, rpt)
  rows = [rows0, rows1]
  sems = [sem0, sem1]
  pltpu.sync_copy(src_i.at[wid], src_v)
  pltpu.sync_copy(dst_i.at[wid], dst_v)
  pltpu.sync_copy(zeros.at[sl], acc.at[sl])
  plsc.subcore_barrier()

  pltpu.async_copy(table.at[src_v.at[0]], rows[0], sems[0])

  @pl.loop(0, npt // NBUF)
  def _(i):
    for b in range(NBUF):
      j = NBUF * i + b
      jnext = jnp.minimum(j + 1, npt - 1)
      pltpu.make_async_copy(table.at[src_v.at[j]], rows[b], sems[b]).wait()
      pltpu.async_copy(table.at[src_v.at[jnext]], rows[1 - b], sems[1 - b])
      pltpu.sync_copy(rows[b], acc.at[dst_v.at[j]], add=True)

  pltpu.make_async_copy(table.at[src_v.at[0]], rows[0], sems[0]).wait()

  plsc.subcore_barrier()
  pltpu.sync_copy(acc.at[sl], out.at[cid].at[sl])


def _make_agg(npt, n_pad, d):
  return pl.kernel(
      functools.partial(_agg_body, npt, n_pad, d),
      out_type=jax.ShapeDtypeStruct((NC, n_pad, d), jnp.float32),
      mesh=_mesh(),
      compiler_params=pltpu.CompilerParams(use_tc_tiling_on_sc=False),
      scratch_types=[
          pltpu.VMEM((npt, K_EDGE), jnp.int32),
          pltpu.VMEM((npt, K_EDGE), jnp.int32),
          pltpu.VMEM((K_EDGE, d), jnp.float32),
          pltpu.VMEM((K_EDGE, d), jnp.float32),
          pltpu.SemaphoreType.DMA,
          pltpu.SemaphoreType.DMA,
          pltpu.VMEM_SHARED((n_pad, d), jnp.float32),
      ],
  )


def _s_block(degp, n, r0):
  dsum = degp[0, :, 0:1] + degp[1, :, 0:1]
  s = jnp.where(dsum > 0, lax.rsqrt(jnp.maximum(dsum, 1e-12)), 0.0)
  rows = r0 + lax.broadcasted_iota(jnp.int32, s.shape, 0)
  return jnp.where(rows < n, s, 0.0)


def _lin_first_body(n, r, x_ref, w_ref, degp_ref, o_ref):
  i = pl.program_id(0)
  s = _s_block(degp_ref[...], n, i * r)
  o_ref[...] = s * jnp.dot(x_ref[...], w_ref[...],
                           preferred_element_type=jnp.float32)


def _lin_mid_body(n, r, p_ref, b_ref, w_ref, degp_ref, o_ref):
  i = pl.program_id(0)
  s = _s_block(degp_ref[...], n, i * r)
  z = s * (p_ref[0] + p_ref[1]) + b_ref[...]
  a = jnp.maximum(z, 0.0)
  o_ref[...] = s * jnp.dot(a, w_ref[...], preferred_element_type=jnp.float32)


def _final_body(n, r, nvalid, p_ref, b_ref, degp_ref, o_ref):
  i = pl.program_id(0)
  s = _s_block(degp_ref[...], n, i * r)
  z = s * (p_ref[0] + p_ref[1]) + b_ref[...]
  col = lax.broadcasted_iota(jnp.int32, z.shape, 1)
  valid = col < nvalid
  zm = jnp.where(valid, z, -jnp.inf)
  m = jnp.max(zm, axis=1, keepdims=True)
  e = jnp.where(valid, jnp.exp(zm - m), 0.0)
  lse = jnp.log(jnp.sum(e, axis=1, keepdims=True))
  o_ref[...] = z - m - lse


_R = 512


def _tc_first(n, n_pad, din, dout):
  grid = n_pad // _R
  return pl.pallas_call(
      functools.partial(_lin_first_body, n, _R),
      grid=(grid,),
      in_specs=[
          pl.BlockSpec((_R, din), lambda i: (i, 0)),
          pl.BlockSpec((din, dout), lambda i: (0, 0)),
          pl.BlockSpec((NC, _R, 16), lambda i: (0, i, 0)),
      ],
      out_specs=pl.BlockSpec((_R, dout), lambda i: (i, 0)),
      out_shape=jax.ShapeDtypeStruct((n_pad, dout), jnp.float32),
  )


def _tc_mid(n, n_pad, din, dout):
  grid = n_pad // _R
  return pl.pallas_call(
      functools.partial(_lin_mid_body, n, _R),
      grid=(grid,),
      in_specs=[
          pl.BlockSpec((NC, _R, din), lambda i: (0, i, 0)),
          pl.BlockSpec((1, din), lambda i: (0, 0)),
          pl.BlockSpec((din, dout), lambda i: (0, 0)),
          pl.BlockSpec((NC, _R, 16), lambda i: (0, i, 0)),
      ],
      out_specs=pl.BlockSpec((_R, dout), lambda i: (i, 0)),
      out_shape=jax.ShapeDtypeStruct((n_pad, dout), jnp.float32),
  )


def _tc_final(n, n_pad, d, nvalid):
  grid = n_pad // _R
  return pl.pallas_call(
      functools.partial(_final_body, n, _R, nvalid),
      grid=(grid,),
      in_specs=[
          pl.BlockSpec((NC, _R, d), lambda i: (0, i, 0)),
          pl.BlockSpec((1, d), lambda i: (0, 0)),
          pl.BlockSpec((NC, _R, 16), lambda i: (0, i, 0)),
      ],
      out_specs=pl.BlockSpec((_R, d), lambda i: (i, 0)),
      out_shape=jax.ShapeDtypeStruct((n_pad, d), jnp.float32),
  )


def kernel(x, edge_index, W1, b1, W2, b2, W3, b3):
  n, in_dim = x.shape
  e = edge_index.shape[1]
  h1 = W1.shape[1]
  h2 = W2.shape[1]
  dout = W3.shape[1]
  dout_p = ((dout + 15) // 16) * 16

  n_pad = ((n + _R - 1) // _R) * _R
  e_tot = e + n
  npt = (e_tot + NW * K_EDGE - 1) // (NW * K_EDGE)
  npt = ((npt + NBUF - 1) // NBUF) * NBUF
  e_pad = NW * npt * K_EDGE

  loop = jnp.arange(n, dtype=jnp.int32)
  src = jnp.concatenate([edge_index[0].astype(jnp.int32), loop])
  dst = jnp.concatenate([edge_index[1].astype(jnp.int32), loop])
  pad = jnp.full((e_pad - e_tot,), n, dtype=jnp.int32)
  src_i = jnp.concatenate([src, pad]).reshape(NW, npt, K_EDGE)
  dst_i = jnp.concatenate([dst, pad]).reshape(NW, npt, K_EDGE)

  x_pad = jnp.pad(x, ((0, n_pad - n), (0, 0)))
  w3p = jnp.pad(W3, ((0, 0), (0, dout_p - dout)))
  b1r = b1.reshape(1, h1)
  b2r = b2.reshape(1, h2)
  b3r = jnp.pad(b3, (0, dout_p - dout)).reshape(1, dout_p)

  ones16 = jnp.ones((K_EDGE, 16), jnp.float32)
  zeros16 = jnp.zeros((n_pad, 16), jnp.float32)

  degp = _make_deg(npt, n_pad)(dst_i, ones16, zeros16)

  t1 = _tc_first(n, n_pad, in_dim, h1)(x_pad, W1, degp)
  t1, sa, da, za = lax.optimization_barrier(
      (t1, src_i, dst_i, jnp.zeros((n_pad, h1), jnp.float32)))
  p1 = _make_agg(npt, n_pad, h1)(t1, sa, da, za)
  t2 = _tc_mid(n, n_pad, h1, h2)(p1, b1r, W2, degp)
  t2, sa, da, za = lax.optimization_barrier(
      (t2, src_i, dst_i, jnp.zeros((n_pad, h2), jnp.float32)))
  p2 = _make_agg(npt, n_pad, h2)(t2, sa, da, za)
  t3 = _tc_mid(n, n_pad, h2, dout_p)(p2, b2r, w3p, degp)
  t3, sa, da, za = lax.optimization_barrier(
      (t3, src_i, dst_i, jnp.zeros((n_pad, dout_p), jnp.float32)))
  p3 = _make_agg(npt, n_pad, dout_p)(t3, sa, da, za)
  o = _tc_final(n, n_pad, dout_p, dout)(p3, b3r, degp)
  return o[:n, :dout]
